# Initial kernel scaffold; baseline (speedup 1.0000x reference)
import jax
import jax.numpy as jnp
from jax.experimental import pallas as pl


def kernel(x, edge_index, W_local, b_local, W_g1, b_g1, W_g2, b_g2, W_fuse, b_fuse):
    raise NotImplementedError("write your pallas kernel here")


# speedup vs baseline: 26.6342x; 26.6342x over previous
import functools

import jax
import jax.numpy as jnp
from jax import lax
from jax.experimental import pallas as pl
from jax.experimental.pallas import tpu as pltpu
from jax.experimental.pallas import tpu_sc as plsc

_N = 10000
_E = 320000
_D = 128
_NP = 10240
_NC = 2
_NS = 16
_CHUNK = 128
_NCH = 80
_NPAIR = _NCH // 2
_EPT = _NCH * _CHUNK
_EP = _NC * _NS * _EPT
_RPT = _NP // _NS


def _mesh():
    return plsc.VectorSubcoreMesh(core_axis_name="c", subcore_axis_name="s")


def _zero_acc(zbuf, acc, r0):
    for i in range(16):
        for j in range(_D // 16):
            zbuf[i, pl.ds(j * 16, 16)] = jnp.zeros((16,), jnp.float32)

    def zero_chunk(k, carry):
        pltpu.sync_copy(zbuf, acc.at[pl.ds(r0 + k * 16, 16)])
        return carry

    lax.fori_loop(0, _RPT // 16, zero_chunk, 0)


def _sc_degree(epk, ones_hbm):
    @functools.partial(
        pl.kernel,
        out_type=jax.ShapeDtypeStruct((_NC, _NP, _D), jnp.float32),
        mesh=_mesh(),
        scratch_types=[
            pltpu.VMEM((2, 2, 2, _CHUNK), jnp.int32),
            pltpu.VMEM((_CHUNK, _D), jnp.float32),
            pltpu.VMEM((16, _D), jnp.float32),
            pltpu.VMEM_SHARED((_NP, _D), jnp.float32),
            pltpu.SemaphoreType.DMA,
            pltpu.SemaphoreType.DMA,
            pltpu.SemaphoreType.DMA,
        ],
    )
    def body(e_hbm, ones_hbm_ref, out_hbm, idxp, ones, zbuf, acc,
             ssem0, ssem1, isem):
        c = lax.axis_index("c")
        s = lax.axis_index("s")
        r0 = s * _RPT
        _zero_acc(zbuf, acc, r0)
        pltpu.sync_copy(ones_hbm_ref, ones)
        plsc.subcore_barrier()

        cbase = (c * _NS + s) * _NCH

        def emit_pair(kidx, p):
            sl = idxp.at[p]
            so = idxp.at[1 - p]
            pltpu.async_copy(ones, acc.at[sl.at[0, 1]], ssem0, add=True)

            @pl.when(kidx > 0)
            def _():
                pltpu.make_async_copy(ones, acc.at[so.at[1, 1]], ssem1).wait()

            @pl.when(kidx < _NPAIR - 1)
            def _():
                pltpu.async_copy(
                    e_hbm.at[pl.ds(cbase + 2 * kidx + 2, 2)], so, isem)

            pltpu.async_copy(ones, acc.at[sl.at[1, 1]], ssem1, add=True)
            pltpu.make_async_copy(ones, acc.at[sl.at[0, 1]], ssem0).wait()

            @pl.when(kidx < _NPAIR - 1)
            def _():
                pltpu.make_async_copy(
                    e_hbm.at[pl.ds(cbase, 2)], so, isem).wait()

        pltpu.sync_copy(e_hbm.at[pl.ds(cbase, 2)], idxp.at[0])

        def pair2(j, carry):
            emit_pair(2 * j, 0)
            emit_pair(2 * j + 1, 1)
            return carry

        lax.fori_loop(0, _NPAIR // 2, pair2, 0)
        pltpu.make_async_copy(ones, acc.at[idxp.at[1, 1, 1]], ssem1).wait()

        plsc.subcore_barrier()
        pltpu.sync_copy(acc.at[pl.ds(r0, _RPT)], out_hbm.at[c, pl.ds(r0, _RPT)])

    return body(epk, ones_hbm)


def _sc_scatter(epk, tab):
    @functools.partial(
        pl.kernel,
        out_type=jax.ShapeDtypeStruct((_NC, _NP, _D), jnp.float32),
        mesh=_mesh(),
        scratch_types=[
            pltpu.VMEM((2, 2, 2, _CHUNK), jnp.int32),
            pltpu.VMEM((_CHUNK, _D), jnp.float32),
            pltpu.VMEM((_CHUNK, _D), jnp.float32),
            pltpu.VMEM((16, _D), jnp.float32),
            pltpu.VMEM_SHARED((_NP, _D), jnp.float32),
            pltpu.SemaphoreType.DMA,
            pltpu.SemaphoreType.DMA,
            pltpu.SemaphoreType.DMA,
            pltpu.SemaphoreType.DMA,
            pltpu.SemaphoreType.DMA,
        ],
    )
    def body(e_hbm, tab_hbm, out_hbm, idxp, rows0, rows1, zbuf, acc,
             gsem0, gsem1, ssem0, ssem1, isem):
        c = lax.axis_index("c")
        s = lax.axis_index("s")
        r0 = s * _RPT
        _zero_acc(zbuf, acc, r0)
        plsc.subcore_barrier()

        cbase = (c * _NS + s) * _NCH

        def emit_pair(kidx, p):
            sl = idxp.at[p]
            so = idxp.at[1 - p]
            pltpu.make_async_copy(tab_hbm.at[sl.at[0, 0]], rows0, gsem0).wait()
            pltpu.async_copy(rows0, acc.at[sl.at[0, 1]], ssem0, add=True)

            @pl.when(kidx > 0)
            def _():
                pltpu.make_async_copy(rows1, acc.at[so.at[1, 1]], ssem1).wait()

            @pl.when(kidx < _NPAIR - 1)
            def _():
                pltpu.async_copy(
                    e_hbm.at[pl.ds(cbase + 2 * kidx + 2, 2)], so, isem)

            pltpu.async_copy(tab_hbm.at[sl.at[1, 0]], rows1, gsem1).wait()
            pltpu.async_copy(rows1, acc.at[sl.at[1, 1]], ssem1, add=True)
            pltpu.make_async_copy(rows0, acc.at[sl.at[0, 1]], ssem0).wait()

            @pl.when(kidx < _NPAIR - 1)
            def _():
                pltpu.make_async_copy(
                    e_hbm.at[pl.ds(cbase, 2)], so, isem).wait()
                pltpu.async_copy(tab_hbm.at[so.at[0, 0]], rows0, gsem0)

        pltpu.sync_copy(e_hbm.at[pl.ds(cbase, 2)], idxp.at[0])
        pltpu.async_copy(tab_hbm.at[idxp.at[0, 0, 0]], rows0, gsem0)

        def pair2(j, carry):
            emit_pair(2 * j, 0)
            emit_pair(2 * j + 1, 1)
            return carry

        lax.fori_loop(0, _NPAIR // 2, pair2, 0)
        pltpu.make_async_copy(rows1, acc.at[idxp.at[1, 1, 1]], ssem1).wait()

        plsc.subcore_barrier()
        pltpu.sync_copy(acc.at[pl.ds(r0, _RPT)], out_hbm.at[c, pl.ds(r0, _RPT)])

    return body(epk, tab)


_BLK = 1024


def _tc_scale_body(dp_ref, xp_ref, xs_ref, dinv_ref):
    dp = dp_ref[...]
    deg = dp[0, :, 0:1] + dp[1, :, 0:1] + 1.0
    dinv = lax.rsqrt(deg)
    xs_ref[...] = xp_ref[...] * dinv
    dinv_ref[...] = jnp.broadcast_to(dinv, (_BLK, 8))


def _tc_scale(deg_parts, xp):
    grid = (_NP // _BLK,)
    return pl.pallas_call(
        _tc_scale_body,
        grid=grid,
        in_specs=[
            pl.BlockSpec((_NC, _BLK, _D), lambda i: (0, i, 0)),
            pl.BlockSpec((_BLK, _D), lambda i: (i, 0)),
        ],
        out_specs=[
            pl.BlockSpec((_BLK, _D), lambda i: (i, 0)),
            pl.BlockSpec((_BLK, 8), lambda i: (i, 0)),
        ],
        out_shape=[
            jax.ShapeDtypeStruct((_NP, _D), jnp.float32),
            jax.ShapeDtypeStruct((_NP, 8), jnp.float32),
        ],
    )(deg_parts, xp)


def _tc_mid_body(dinv_ref, acc_ref, xs_ref, w_ref, b_ref, z_ref, g1s_ref):
    dinv = dinv_ref[:, 0:1]
    acc = acc_ref[...]
    z = dinv * (acc[0] + acc[1] + xs_ref[...])
    g1 = jnp.maximum(
        jnp.dot(z, w_ref[...], precision=lax.Precision.HIGHEST) + b_ref[...], 0.0
    )
    z_ref[...] = z
    g1s_ref[...] = dinv * g1


def _tc_mid(dinv8, accx, xs, w_g1, b_g1):
    grid = (_NP // _BLK,)
    full = lambda i: (0, 0)
    return pl.pallas_call(
        _tc_mid_body,
        grid=grid,
        in_specs=[
            pl.BlockSpec((_BLK, 8), lambda i: (i, 0)),
            pl.BlockSpec((_NC, _BLK, _D), lambda i: (0, i, 0)),
            pl.BlockSpec((_BLK, _D), lambda i: (i, 0)),
            pl.BlockSpec((_D, _D), full),
            pl.BlockSpec((1, _D), full),
        ],
        out_specs=[
            pl.BlockSpec((_BLK, _D), lambda i: (i, 0)),
            pl.BlockSpec((_BLK, _D), lambda i: (i, 0)),
        ],
        out_shape=[
            jax.ShapeDtypeStruct((_NP, _D), jnp.float32),
            jax.ShapeDtypeStruct((_NP, _D), jnp.float32),
        ],
    )(dinv8, accx, xs, w_g1, b_g1)


def _tc_final_body(dinv_ref, accg_ref, g1s_ref, z_ref, wl_ref, bl_ref, w2_ref,
                   b2_ref, wf1_ref, wf2_ref, bf_ref, out_ref):
    dinv = dinv_ref[:, 0:1]
    accg = accg_ref[...]
    hi = lax.Precision.HIGHEST
    pg = dinv * (accg[0] + accg[1] + g1s_ref[...])
    g2 = jnp.maximum(jnp.dot(pg, w2_ref[...], precision=hi) + b2_ref[...], 0.0)
    loc = jnp.maximum(
        jnp.dot(z_ref[...], wl_ref[...], precision=hi) + bl_ref[...], 0.0
    )
    out_ref[...] = (
        jnp.dot(loc, wf1_ref[...], precision=hi)
        + jnp.dot(g2, wf2_ref[...], precision=hi)
        + bf_ref[...]
    )


def _tc_final(dinv8, accg, g1s, z, w_l, b_l, w_2, b_2, wf1, wf2, b_f):
    grid = (_NP // _BLK,)
    full = lambda i: (0, 0)
    return pl.pallas_call(
        _tc_final_body,
        grid=grid,
        in_specs=[
            pl.BlockSpec((_BLK, 8), lambda i: (i, 0)),
            pl.BlockSpec((_NC, _BLK, _D), lambda i: (0, i, 0)),
            pl.BlockSpec((_BLK, _D), lambda i: (i, 0)),
            pl.BlockSpec((_BLK, _D), lambda i: (i, 0)),
            pl.BlockSpec((_D, _D), full),
            pl.BlockSpec((1, _D), full),
            pl.BlockSpec((_D, _D), full),
            pl.BlockSpec((1, _D), full),
            pl.BlockSpec((_D, _D), full),
            pl.BlockSpec((_D, _D), full),
            pl.BlockSpec((1, _D), full),
        ],
        out_specs=pl.BlockSpec((_BLK, _D), lambda i: (i, 0)),
        out_shape=jax.ShapeDtypeStruct((_NP, _D), jnp.float32),
    )(dinv8, accg, g1s, z, w_l, b_l, w_2, b_2, wf1, wf2, b_f)


def kernel(x, edge_index, W_local, b_local, W_g1, b_g1, W_g2, b_g2, W_fuse, b_fuse):
    pad_n = _EP - _E
    pad_idx = (_N + (jnp.arange(pad_n, dtype=jnp.int32) % (_NP - _N))).astype(
        jnp.int32
    )
    src = jnp.concatenate([edge_index[0], pad_idx])
    dst = jnp.concatenate([edge_index[1], pad_idx])
    epk = jnp.concatenate(
        [src.reshape(-1, 1, _CHUNK), dst.reshape(-1, 1, _CHUNK)], axis=1
    )
    xp = jnp.zeros((_NP, _D), jnp.float32).at[:_N].set(x)
    ones = jnp.ones((_CHUNK, _D), jnp.float32)

    deg_parts = _sc_degree(epk, ones)
    xs, dinv8 = _tc_scale(deg_parts, xp)
    accx = _sc_scatter(epk, xs)
    z, g1s = _tc_mid(dinv8, accx, xs, W_g1, b_g1.reshape(1, _D))
    accg = _sc_scatter(epk, g1s)
    out = _tc_final(
        dinv8, accg, g1s, z,
        W_local, b_local.reshape(1, _D),
        W_g2, b_g2.reshape(1, _D),
        W_fuse[:_D], W_fuse[_D:], b_fuse.reshape(1, _D),
    )
    return out[:_N]
